# baseline (speedup 1.0000x reference)
import functools

import jax
import jax.numpy as jnp
from jax import lax
from jax.experimental import pallas as pl
from jax.experimental.pallas import tpu as pltpu
from jax.experimental.pallas import tpu_sc as plsc

_B = 4
_S = 4096
_D = 1024
_NC = 2
_NS = 16
_NW = _NC * _NS
_PPW = _S // _NW
_C = 8
_NCHUNK = _PPW // _C
_LANES = 16


def _body(tokens_hbm, table_hbm, pos_hbm, out_hbm,
          idx_v, pos_v, rows_v, sem_in0, sem_in1, sem_out0, sem_out1):
    wid = lax.axis_index("s") * _NC + lax.axis_index("c")
    p0 = wid * _PPW
    sem_in = (sem_in0, sem_in1)
    sem_out = (sem_out0, sem_out1)

    for b in range(_B):
        pltpu.sync_copy(tokens_hbm.at[b, pl.ds(p0, _PPW)], idx_v.at[b])

    def issue_in(ci, p):
        s0 = p0 + ci * _C
        c0 = ci * _C
        pltpu.async_copy(pos_hbm.at[pl.ds(s0, _C)], pos_v.at[p], sem_in[p])
        for b in range(_B):
            pltpu.async_copy(
                table_hbm.at[idx_v.at[b, pl.ds(c0, _C)]],
                rows_v.at[p, pl.ds(b * _C, _C)],
                sem_in[p],
            )

    def wait_in(p):
        pltpu.make_async_copy(pos_hbm.at[pl.ds(0, _C)], pos_v.at[p],
                              sem_in[p]).wait()
        for b in range(_B):
            pltpu.make_async_copy(
                pos_hbm.at[pl.ds(0, _C)],
                rows_v.at[p, pl.ds(b * _C, _C)],
                sem_in[p],
            ).wait()

    def issue_out(ci, p):
        s0 = p0 + ci * _C
        for b in range(_B):
            pltpu.async_copy(
                rows_v.at[p, pl.ds(b * _C, _C)],
                out_hbm.at[b, pl.ds(s0, _C)],
                sem_out[p],
            )

    def wait_out(p):
        for b in range(_B):
            pltpu.make_async_copy(
                pos_hbm.at[pl.ds(0, _C)],
                rows_v.at[p, pl.ds(b * _C, _C)],
                sem_out[p],
            ).wait()

    def add_chunk(p):
        def addloop(i, c2):
            off = i * _LANES
            for c in range(_C):
                pv = pos_v[p, c, pl.ds(off, _LANES)]
                for b in range(_B):
                    r = b * _C + c
                    rows_v[p, r, pl.ds(off, _LANES)] = (
                        rows_v[p, r, pl.ds(off, _LANES)] + pv
                    )
            return c2

        lax.fori_loop(0, _D // _LANES, addloop, 0, unroll=2)

    issue_in(0, 0)

    def pair(i, carry):
        ci0 = 2 * i
        ci1 = ci0 + 1
        wait_in(0)

        @pl.when(i > 0)
        def _():
            wait_out(1)

        issue_in(ci1, 1)
        add_chunk(0)
        issue_out(ci0, 0)
        wait_in(1)
        wait_out(0)

        @pl.when(ci1 + 1 < _NCHUNK)
        def _():
            issue_in(ci1 + 1, 0)

        add_chunk(1)
        issue_out(ci1, 1)
        return carry

    lax.fori_loop(0, _NCHUNK // 2, pair, 0)
    wait_out(1)


@jax.jit
def _emb(tokens, table, pos2d):
    mesh = plsc.VectorSubcoreMesh(core_axis_name="c", subcore_axis_name="s")
    return pl.kernel(
        _body,
        out_type=jax.ShapeDtypeStruct((_B, _S, _D), jnp.float32),
        mesh=mesh,
        scratch_types=[
            pltpu.VMEM((_B, _PPW), jnp.int32),
            pltpu.VMEM((2, _C, _D), jnp.float32),
            pltpu.VMEM((2, _B * _C, _D), jnp.float32),
            pltpu.SemaphoreType.DMA,
            pltpu.SemaphoreType.DMA,
            pltpu.SemaphoreType.DMA,
            pltpu.SemaphoreType.DMA,
        ],
    )(tokens, table, pos2d)


def kernel(tokens, table, pos):
    tokens = tokens.astype(jnp.int32)
    pos2d = pos.reshape(pos.shape[1], pos.shape[2])[: tokens.shape[1]]
    return _emb(tokens, table, pos2d)


# speedup vs baseline: 2.6617x; 2.6617x over previous
import functools

import jax
import jax.numpy as jnp
from jax import lax
from jax.experimental import pallas as pl
from jax.experimental.pallas import tpu as pltpu
from jax.experimental.pallas import tpu_sc as plsc

_B = 4
_S = 4096
_D = 1024
_NC = 2
_NS = 16
_NW = _NC * _NS
_PPW = _S // _NW
_C = 8
_NCHUNK = _PPW // _C
_LANES = 16


def _body(tokens_hbm, table_hbm, pos_hbm, out_hbm,
          idx_v, pos_v, rows_v, sem_in0, sem_in1, sem_out0, sem_out1):
    wid = lax.axis_index("s") * _NC + lax.axis_index("c")
    p0 = wid * _PPW
    sem_in = (sem_in0, sem_in1)
    sem_out = (sem_out0, sem_out1)

    for b in range(_B):
        pltpu.sync_copy(tokens_hbm.at[b, pl.ds(p0, _PPW)], idx_v.at[b])

    def issue_in(ci, p):
        s0 = p0 + ci * _C
        c0 = ci * _C
        pltpu.async_copy(pos_hbm.at[pl.ds(s0, _C)], pos_v.at[p], sem_in[p])
        for b in range(_B):
            pltpu.async_copy(
                table_hbm.at[idx_v.at[b, pl.ds(c0, _C)]],
                rows_v.at[p, pl.ds(b * _C, _C)],
                sem_in[p],
            )

    def wait_in(p):
        pltpu.make_async_copy(pos_hbm.at[pl.ds(0, _C)], pos_v.at[p],
                              sem_in[p]).wait()
        for b in range(_B):
            pltpu.make_async_copy(
                pos_hbm.at[pl.ds(0, _C)],
                rows_v.at[p, pl.ds(b * _C, _C)],
                sem_in[p],
            ).wait()

    def issue_out(ci, p):
        s0 = p0 + ci * _C
        for b in range(_B):
            pltpu.async_copy(
                rows_v.at[p, pl.ds(b * _C, _C)],
                out_hbm.at[b, pl.ds(s0, _C)],
                sem_out[p],
            )

    def wait_out(p):
        for b in range(_B):
            pltpu.make_async_copy(
                pos_hbm.at[pl.ds(0, _C)],
                rows_v.at[p, pl.ds(b * _C, _C)],
                sem_out[p],
            ).wait()

    def add_chunk(p):
        @plsc.parallel_loop(0, _C * (_D // _LANES), unroll=2)
        def _(t):
            c = t >> 6
            off = (t & (_D // _LANES - 1)) * _LANES
            pv = pos_v[p, c, pl.ds(off, _LANES)]
            for b in range(_B):
                r = b * _C + c
                rows_v[p, r, pl.ds(off, _LANES)] = (
                    rows_v[p, r, pl.ds(off, _LANES)] + pv
                )

    issue_in(0, 0)

    def pair(i, carry):
        ci0 = 2 * i
        ci1 = ci0 + 1
        wait_in(0)

        @pl.when(i > 0)
        def _():
            wait_out(1)

        issue_in(ci1, 1)
        add_chunk(0)
        issue_out(ci0, 0)
        wait_in(1)
        wait_out(0)

        @pl.when(ci1 + 1 < _NCHUNK)
        def _():
            issue_in(ci1 + 1, 0)

        add_chunk(1)
        issue_out(ci1, 1)
        return carry

    lax.fori_loop(0, _NCHUNK // 2, pair, 0)
    wait_out(1)


@jax.jit
def _emb(tokens, table, pos2d):
    mesh = plsc.VectorSubcoreMesh(core_axis_name="c", subcore_axis_name="s")
    return pl.kernel(
        _body,
        out_type=jax.ShapeDtypeStruct((_B, _S, _D), jnp.float32),
        mesh=mesh,
        scratch_types=[
            pltpu.VMEM((_B, _PPW), jnp.int32),
            pltpu.VMEM((2, _C, _D), jnp.float32),
            pltpu.VMEM((2, _B * _C, _D), jnp.float32),
            pltpu.SemaphoreType.DMA,
            pltpu.SemaphoreType.DMA,
            pltpu.SemaphoreType.DMA,
            pltpu.SemaphoreType.DMA,
        ],
    )(tokens, table, pos2d)


def kernel(tokens, table, pos):
    tokens = tokens.astype(jnp.int32)
    pos2d = pos.reshape(pos.shape[1], pos.shape[2])[: tokens.shape[1]]
    return _emb(tokens, table, pos2d)
